# baseline (speedup 1.0000x reference)
import functools

import jax
import jax.numpy as jnp
from jax import lax
from jax.experimental import pallas as pl
from jax.experimental.pallas import tpu as pltpu
from jax.experimental.pallas import tpu_sc as plsc

N_NODES = 10000
N_EDGES = 10000
NNZ = 320000
D_IN = 128
D_OUT = 128
HALF = 64

NS = 16
ROWS = 10112
DUMMY = 10016
ROWS_PER_TILE = ROWS // NS
CHUNK = 128
NBUF = 4
NCHUNK = 160
NGROUP = NCHUNK // NBUF
PER_TILE = NCHUNK * CHUNK
NNZ_PAD = PER_TILE * NS


def _mm_body(x_ref, w_ref, oa_ref, ob_ref):
    y = jnp.dot(x_ref[...], w_ref[...], preferred_element_type=jnp.float32)
    oa_ref[...] = y[:, :HALF]
    ob_ref[...] = y[:, HALF:]


def _matmul_halves(x0p, weight):
    rb = ROWS // 4
    return pl.pallas_call(
        _mm_body,
        grid=(4,),
        in_specs=[
            pl.BlockSpec((rb, D_IN), lambda i: (i, 0)),
            pl.BlockSpec((D_IN, D_OUT), lambda i: (0, 0)),
        ],
        out_specs=[
            pl.BlockSpec((rb, HALF), lambda i: (i, 0)),
            pl.BlockSpec((rb, HALF), lambda i: (i, 0)),
        ],
        out_shape=[
            jax.ShapeDtypeStruct((ROWS, HALF), jnp.float32),
            jax.ShapeDtypeStruct((ROWS, HALF), jnp.float32),
        ],
    )(x0p, weight)


def _hop(table, gidx, sidx, acc, bufs, gi_v, si_v, dsem, isem, zero_hbm):
    pltpu.sync_copy(gidx.at[0], gi_v.at[0])
    pltpu.sync_copy(sidx.at[0], si_v.at[0])
    for b in range(NBUF):
        pltpu.async_copy(table.at[gi_v.at[0].at[b]], bufs.at[b], dsem.at[b])
    pltpu.async_copy(gidx.at[1], gi_v.at[1], isem)
    pltpu.async_copy(sidx.at[1], si_v.at[1], isem)

    def group(g, carry):
        p = g % 2
        q = (g + 1) % 2
        pltpu.make_async_copy(gidx.at[0], gi_v.at[q], isem).wait()
        pltpu.make_async_copy(sidx.at[0], si_v.at[q], isem).wait()
        for b in range(NBUF):
            pltpu.make_async_copy(zero_hbm.at[pl.ds(0, CHUNK)], bufs.at[b],
                                  dsem.at[b]).wait()
            pltpu.sync_copy(bufs.at[b], acc.at[si_v.at[p].at[b]], add=True)
            pltpu.async_copy(table.at[gi_v.at[q].at[b]], bufs.at[b],
                             dsem.at[b])
        gn = jnp.minimum(g + 2, NGROUP - 1)
        pltpu.async_copy(gidx.at[gn], gi_v.at[p], isem)
        pltpu.async_copy(sidx.at[gn], si_v.at[p], isem)
        return carry

    lax.fori_loop(0, NGROUP, group, 0)
    for b in range(NBUF):
        pltpu.make_async_copy(zero_hbm.at[pl.ds(0, CHUNK)], bufs.at[b],
                              dsem.at[b]).wait()
    pltpu.make_async_copy(gidx.at[0], gi_v.at[0], isem).wait()
    pltpu.make_async_copy(sidx.at[0], si_v.at[0], isem).wait()


def _sc_body(xwa, xwb, node_hbm, edge_hbm, zero_hbm, out_a, out_b,
             gi_v, si_v, bufs, acc_m, acc_o, dsem, isem):
    c = lax.axis_index("c")
    s = lax.axis_index("s")
    r0 = s * ROWS_PER_TILE
    node_s = node_hbm.at[s]
    edge_s = edge_hbm.at[s]

    pltpu.sync_copy(zero_hbm.at[pl.ds(r0, ROWS_PER_TILE)],
                    acc_m.at[pl.ds(r0, ROWS_PER_TILE)])
    pltpu.sync_copy(zero_hbm.at[pl.ds(r0, ROWS_PER_TILE)],
                    acc_o.at[pl.ds(r0, ROWS_PER_TILE)])
    plsc.subcore_barrier()

    @pl.when(c == 0)
    def _():
        _hop(xwa, node_s, edge_s, acc_m, bufs, gi_v, si_v, dsem, isem,
             zero_hbm)

    @pl.when(c == 1)
    def _():
        _hop(xwb, node_s, edge_s, acc_m, bufs, gi_v, si_v, dsem, isem,
             zero_hbm)

    plsc.subcore_barrier()

    @pl.when(c == 0)
    def _():
        pltpu.sync_copy(acc_m.at[pl.ds(r0, ROWS_PER_TILE)],
                        out_a.at[pl.ds(r0, ROWS_PER_TILE)])

    @pl.when(c == 1)
    def _():
        pltpu.sync_copy(acc_m.at[pl.ds(r0, ROWS_PER_TILE)],
                        out_b.at[pl.ds(r0, ROWS_PER_TILE)])

    plsc.subcore_barrier()

    @pl.when(c == 0)
    def _():
        _hop(out_a, edge_s, node_s, acc_o, bufs, gi_v, si_v, dsem, isem,
             zero_hbm)

    @pl.when(c == 1)
    def _():
        _hop(out_b, edge_s, node_s, acc_o, bufs, gi_v, si_v, dsem, isem,
             zero_hbm)

    plsc.subcore_barrier()

    @pl.when(c == 0)
    def _():
        pltpu.sync_copy(acc_o.at[pl.ds(r0, ROWS_PER_TILE)],
                        out_a.at[pl.ds(r0, ROWS_PER_TILE)])

    @pl.when(c == 1)
    def _():
        pltpu.sync_copy(acc_o.at[pl.ds(r0, ROWS_PER_TILE)],
                        out_b.at[pl.ds(r0, ROWS_PER_TILE)])


_sc_call = pl.kernel(
    _sc_body,
    out_type=[
        jax.ShapeDtypeStruct((ROWS, HALF), jnp.float32),
        jax.ShapeDtypeStruct((ROWS, HALF), jnp.float32),
    ],
    mesh=plsc.VectorSubcoreMesh(core_axis_name="c", subcore_axis_name="s"),
    scratch_types=[
        pltpu.VMEM((2, NBUF, CHUNK), jnp.int32),
        pltpu.VMEM((2, NBUF, CHUNK), jnp.int32),
        pltpu.VMEM((NBUF, CHUNK, HALF), jnp.float32),
        pltpu.VMEM_SHARED((ROWS, HALF), jnp.float32),
        pltpu.VMEM_SHARED((ROWS, HALF), jnp.float32),
        pltpu.SemaphoreType.DMA((NBUF,)),
        pltpu.SemaphoreType.DMA,
    ],
    compiler_params=pltpu.CompilerParams(use_tc_tiling_on_sc=False),
)


@jax.jit
def kernel(x_0, node_idx, edge_idx, weight):
    x0p = jnp.zeros((ROWS, D_IN), jnp.float32).at[:N_NODES].set(x_0)
    pad = jnp.full((NNZ_PAD - NNZ,), DUMMY, jnp.int32)
    node3 = jnp.concatenate([node_idx, pad]).reshape(NS, NGROUP, NBUF, CHUNK)
    edge3 = jnp.concatenate([edge_idx, pad]).reshape(NS, NGROUP, NBUF, CHUNK)
    zeros = jnp.zeros((ROWS, HALF), jnp.float32)
    xwa, xwb = _matmul_halves(x0p, weight)
    out_a, out_b = _sc_call(xwa, xwb, node3, edge3, zeros)
    return jnp.concatenate([out_a[:N_NODES], out_b[:N_NODES]], axis=1)


# speedup vs baseline: 1.6889x; 1.6889x over previous
import functools

import jax
import jax.numpy as jnp
from jax import lax
from jax.experimental import pallas as pl
from jax.experimental.pallas import tpu as pltpu
from jax.experimental.pallas import tpu_sc as plsc

N_NODES = 10000
N_EDGES = 10000
NNZ = 320000
D_IN = 128
D_OUT = 128
HALF = 64

NS = 16
ROWS = 10112
DUMMY = 10016
ROWS_PER_TILE = ROWS // NS
CHUNK = 128
NBUF = 4
NCHUNK = 160
NGROUP = NCHUNK // NBUF
PER_TILE = NCHUNK * CHUNK
NNZ_PAD = PER_TILE * NS


def _mm_body(x_ref, w_ref, oa_ref, ob_ref):
    y = jnp.dot(x_ref[...], w_ref[...], preferred_element_type=jnp.float32)
    oa_ref[...] = y[:, :HALF]
    ob_ref[...] = y[:, HALF:]


def _matmul_halves(x0p, weight):
    rb = ROWS // 4
    return pl.pallas_call(
        _mm_body,
        grid=(4,),
        in_specs=[
            pl.BlockSpec((rb, D_IN), lambda i: (i, 0)),
            pl.BlockSpec((D_IN, D_OUT), lambda i: (0, 0)),
        ],
        out_specs=[
            pl.BlockSpec((rb, HALF), lambda i: (i, 0)),
            pl.BlockSpec((rb, HALF), lambda i: (i, 0)),
        ],
        out_shape=[
            jax.ShapeDtypeStruct((ROWS, HALF), jnp.float32),
            jax.ShapeDtypeStruct((ROWS, HALF), jnp.float32),
        ],
    )(x0p, weight)


def _hop(table, gidx, sidx, acc, bufs, gi_v, si_v, dsem, isem, zero_hbm):
    pltpu.sync_copy(gidx.at[0], gi_v.at[0])
    pltpu.sync_copy(sidx.at[0], si_v.at[0])
    for b in range(NBUF):
        pltpu.async_copy(table.at[gi_v.at[0].at[b]], bufs.at[b], dsem.at[b])
    pltpu.async_copy(gidx.at[1], gi_v.at[1], isem)
    pltpu.async_copy(sidx.at[1], si_v.at[1], isem)

    def group(g, carry):
        p = g % 2
        q = (g + 1) % 2
        pltpu.make_async_copy(gidx.at[0], gi_v.at[q], isem).wait()
        pltpu.make_async_copy(sidx.at[0], si_v.at[q], isem).wait()
        for b in range(NBUF):
            pltpu.make_async_copy(zero_hbm.at[pl.ds(0, CHUNK)], bufs.at[b],
                                  dsem.at[b]).wait()
            pltpu.sync_copy(bufs.at[b], acc.at[si_v.at[p].at[b]], add=True)
            pltpu.async_copy(table.at[gi_v.at[q].at[b]], bufs.at[b],
                             dsem.at[b])
        gn = jnp.minimum(g + 2, NGROUP - 1)
        pltpu.async_copy(gidx.at[gn], gi_v.at[p], isem)
        pltpu.async_copy(sidx.at[gn], si_v.at[p], isem)
        return carry

    lax.fori_loop(0, NGROUP, group, 0)
    for b in range(NBUF):
        pltpu.make_async_copy(zero_hbm.at[pl.ds(0, CHUNK)], bufs.at[b],
                              dsem.at[b]).wait()
    pltpu.make_async_copy(gidx.at[0], gi_v.at[0], isem).wait()
    pltpu.make_async_copy(sidx.at[0], si_v.at[0], isem).wait()


def _sc_body(xwa, xwb, node_hbm, edge_hbm, zero_hbm, out_a, out_b,
             gi_v, si_v, bufs, xwsp, acc_m, dsem, isem):
    c = lax.axis_index("c")
    s = lax.axis_index("s")
    r0 = s * ROWS_PER_TILE
    rows = pl.ds(r0, ROWS_PER_TILE)
    node_s = node_hbm.at[s]
    edge_s = edge_hbm.at[s]

    @pl.when(c == 0)
    def _():
        pltpu.sync_copy(xwa.at[rows], xwsp.at[rows])

    @pl.when(c == 1)
    def _():
        pltpu.sync_copy(xwb.at[rows], xwsp.at[rows])

    pltpu.sync_copy(zero_hbm.at[rows], acc_m.at[rows])
    plsc.subcore_barrier()

    _hop(xwsp, node_s, edge_s, acc_m, bufs, gi_v, si_v, dsem, isem, zero_hbm)
    plsc.subcore_barrier()

    pltpu.sync_copy(zero_hbm.at[rows], xwsp.at[rows])
    plsc.subcore_barrier()

    _hop(acc_m, edge_s, node_s, xwsp, bufs, gi_v, si_v, dsem, isem, zero_hbm)
    plsc.subcore_barrier()

    @pl.when(c == 0)
    def _():
        pltpu.sync_copy(xwsp.at[rows], out_a.at[rows])

    @pl.when(c == 1)
    def _():
        pltpu.sync_copy(xwsp.at[rows], out_b.at[rows])


_sc_call = pl.kernel(
    _sc_body,
    out_type=[
        jax.ShapeDtypeStruct((ROWS, HALF), jnp.float32),
        jax.ShapeDtypeStruct((ROWS, HALF), jnp.float32),
    ],
    mesh=plsc.VectorSubcoreMesh(core_axis_name="c", subcore_axis_name="s"),
    scratch_types=[
        pltpu.VMEM((2, NBUF, CHUNK), jnp.int32),
        pltpu.VMEM((2, NBUF, CHUNK), jnp.int32),
        pltpu.VMEM((NBUF, CHUNK, HALF), jnp.float32),
        pltpu.VMEM_SHARED((ROWS, HALF), jnp.float32),
        pltpu.VMEM_SHARED((ROWS, HALF), jnp.float32),
        pltpu.SemaphoreType.DMA((NBUF,)),
        pltpu.SemaphoreType.DMA,
    ],
    compiler_params=pltpu.CompilerParams(use_tc_tiling_on_sc=False),
)


@jax.jit
def kernel(x_0, node_idx, edge_idx, weight):
    x0p = jnp.zeros((ROWS, D_IN), jnp.float32).at[:N_NODES].set(x_0)
    pad = jnp.full((NNZ_PAD - NNZ,), DUMMY, jnp.int32)
    node3 = jnp.concatenate([node_idx, pad]).reshape(NS, NGROUP, NBUF, CHUNK)
    edge3 = jnp.concatenate([edge_idx, pad]).reshape(NS, NGROUP, NBUF, CHUNK)
    zeros = jnp.zeros((ROWS, HALF), jnp.float32)
    xwa, xwb = _matmul_halves(x0p, weight)
    out_a, out_b = _sc_call(xwa, xwb, node3, edge3, zeros)
    return jnp.concatenate([out_a[:N_NODES], out_b[:N_NODES]], axis=1)


# speedup vs baseline: 1.7969x; 1.0639x over previous
import functools

import jax
import jax.numpy as jnp
from jax import lax
from jax.experimental import pallas as pl
from jax.experimental.pallas import tpu as pltpu
from jax.experimental.pallas import tpu_sc as plsc

N_NODES = 10000
N_EDGES = 10000
NNZ = 320000
D_IN = 128
D_OUT = 128
HALF = 64

NS = 16
ROWS = 10112
DUMMY = 10016
ROWS_PER_TILE = ROWS // NS
CHUNK = 64
CPG = 4
NSLOT = 2 * CPG
NCHUNK = 320
NGROUP = NCHUNK // CPG
PER_TILE = NCHUNK * CHUNK
NNZ_PAD = PER_TILE * NS


def _mm_body(x_ref, w_ref, oa_ref, ob_ref):
    y = jnp.dot(x_ref[...], w_ref[...], preferred_element_type=jnp.float32)
    oa_ref[...] = y[:, :HALF]
    ob_ref[...] = y[:, HALF:]


def _matmul_halves(x0p, weight):
    rb = ROWS // 4
    return pl.pallas_call(
        _mm_body,
        grid=(4,),
        in_specs=[
            pl.BlockSpec((rb, D_IN), lambda i: (i, 0)),
            pl.BlockSpec((D_IN, D_OUT), lambda i: (0, 0)),
        ],
        out_specs=[
            pl.BlockSpec((rb, HALF), lambda i: (i, 0)),
            pl.BlockSpec((rb, HALF), lambda i: (i, 0)),
        ],
        out_shape=[
            jax.ShapeDtypeStruct((ROWS, HALF), jnp.float32),
            jax.ShapeDtypeStruct((ROWS, HALF), jnp.float32),
        ],
    )(x0p, weight)


def _hop(table, gidx, sidx, acc, bufs, gi_v, si_v, dsem, ssem, isem,
         zero_hbm):
    def dwait(sl):
        pltpu.make_async_copy(zero_hbm.at[pl.ds(0, CHUNK)], bufs.at[sl],
                              dsem.at[sl]).wait()

    def swait(sl):
        pltpu.make_async_copy(zero_hbm.at[pl.ds(0, CHUNK)], bufs.at[sl],
                              ssem.at[sl]).wait()

    def iwait(dst):
        pltpu.make_async_copy(gidx.at[0], dst, isem).wait()

    pltpu.sync_copy(gidx.at[0], gi_v.at[0])
    pltpu.sync_copy(sidx.at[0], si_v.at[0])
    pltpu.async_copy(gidx.at[1], gi_v.at[1], isem)
    pltpu.async_copy(sidx.at[1], si_v.at[1], isem)
    for b in range(CPG):
        pltpu.async_copy(table.at[gi_v.at[0].at[b]], bufs.at[b], dsem.at[b])

    iwait(gi_v.at[1])
    iwait(si_v.at[1])
    for b in range(CPG):
        dwait(b)
        pltpu.async_copy(bufs.at[b], acc.at[si_v.at[0].at[b]], ssem.at[b],
                         add=True)
    for b in range(CPG):
        pltpu.async_copy(table.at[gi_v.at[1].at[b]], bufs.at[CPG + b],
                         dsem.at[CPG + b])
    pltpu.async_copy(gidx.at[2], gi_v.at[0], isem)
    pltpu.async_copy(sidx.at[2], si_v.at[0], isem)

    def group(g, carry):
        p = g % 2
        q = (g + 1) % 2
        iwait(gi_v.at[q])
        iwait(si_v.at[q])
        for b in range(CPG):
            sl = CPG * p + b
            dwait(sl)
            pltpu.async_copy(bufs.at[sl], acc.at[si_v.at[p].at[b]],
                             ssem.at[sl], add=True)
        for b in range(CPG):
            sl = CPG * q + b
            swait(sl)
            pltpu.async_copy(table.at[gi_v.at[q].at[b]], bufs.at[sl],
                             dsem.at[sl])
        gn = jnp.minimum(g + 2, NGROUP - 1)
        pltpu.async_copy(gidx.at[gn], gi_v.at[p], isem)
        pltpu.async_copy(sidx.at[gn], si_v.at[p], isem)
        return carry

    lax.fori_loop(1, NGROUP, group, 0)

    pl_ = (NGROUP - 1) % 2
    for b in range(CPG):
        swait(CPG * pl_ + b)
    for b in range(CPG):
        dwait(CPG * (1 - pl_) + b)
    iwait(gi_v.at[1 - pl_])
    iwait(si_v.at[1 - pl_])


def _sc_body(xwa, xwb, node_hbm, edge_hbm, zero_hbm, out_a, out_b,
             gi_v, si_v, bufs, xwsp, acc_m, dsem, ssem, isem):
    c = lax.axis_index("c")
    s = lax.axis_index("s")
    r0 = s * ROWS_PER_TILE
    rows = pl.ds(r0, ROWS_PER_TILE)
    node_s = node_hbm.at[s]
    edge_s = edge_hbm.at[s]

    @pl.when(c == 0)
    def _():
        pltpu.sync_copy(xwa.at[rows], xwsp.at[rows])

    @pl.when(c == 1)
    def _():
        pltpu.sync_copy(xwb.at[rows], xwsp.at[rows])

    pltpu.sync_copy(zero_hbm.at[rows], acc_m.at[rows])
    plsc.subcore_barrier()

    _hop(xwsp, node_s, edge_s, acc_m, bufs, gi_v, si_v, dsem, ssem, isem,
         zero_hbm)
    plsc.subcore_barrier()

    pltpu.sync_copy(zero_hbm.at[rows], xwsp.at[rows])
    plsc.subcore_barrier()

    _hop(acc_m, edge_s, node_s, xwsp, bufs, gi_v, si_v, dsem, ssem, isem,
         zero_hbm)
    plsc.subcore_barrier()

    @pl.when(c == 0)
    def _():
        pltpu.sync_copy(xwsp.at[rows], out_a.at[rows])

    @pl.when(c == 1)
    def _():
        pltpu.sync_copy(xwsp.at[rows], out_b.at[rows])


_sc_call = pl.kernel(
    _sc_body,
    out_type=[
        jax.ShapeDtypeStruct((ROWS, HALF), jnp.float32),
        jax.ShapeDtypeStruct((ROWS, HALF), jnp.float32),
    ],
    mesh=plsc.VectorSubcoreMesh(core_axis_name="c", subcore_axis_name="s"),
    scratch_types=[
        pltpu.VMEM((2, CPG, CHUNK), jnp.int32),
        pltpu.VMEM((2, CPG, CHUNK), jnp.int32),
        pltpu.VMEM((NSLOT, CHUNK, HALF), jnp.float32),
        pltpu.VMEM_SHARED((ROWS, HALF), jnp.float32),
        pltpu.VMEM_SHARED((ROWS, HALF), jnp.float32),
        pltpu.SemaphoreType.DMA((NSLOT,)),
        pltpu.SemaphoreType.DMA((NSLOT,)),
        pltpu.SemaphoreType.DMA,
    ],
    compiler_params=pltpu.CompilerParams(use_tc_tiling_on_sc=False),
)


@jax.jit
def kernel(x_0, node_idx, edge_idx, weight):
    x0p = jnp.zeros((ROWS, D_IN), jnp.float32).at[:N_NODES].set(x_0)
    pad = jnp.full((NNZ_PAD - NNZ,), DUMMY, jnp.int32)
    node3 = jnp.concatenate([node_idx, pad]).reshape(NS, NGROUP, CPG, CHUNK)
    edge3 = jnp.concatenate([edge_idx, pad]).reshape(NS, NGROUP, CPG, CHUNK)
    zeros = jnp.zeros((ROWS, HALF), jnp.float32)
    xwa, xwb = _matmul_halves(x0p, weight)
    out_a, out_b = _sc_call(xwa, xwb, node3, edge3, zeros)
    return jnp.concatenate([out_a[:N_NODES], out_b[:N_NODES]], axis=1)


# speedup vs baseline: 1.7995x; 1.0015x over previous
import functools

import jax
import jax.numpy as jnp
from jax import lax
from jax.experimental import pallas as pl
from jax.experimental.pallas import tpu as pltpu
from jax.experimental.pallas import tpu_sc as plsc

N_NODES = 10000
N_EDGES = 10000
NNZ = 320000
D_IN = 128
D_OUT = 128
HALF = 64

NS = 16
ROWS = 10112
DUMMY = 10016
ROWS_PER_TILE = ROWS // NS
CHUNK = 64
CPG = 4
NSLOT = 2 * CPG
NCHUNK = 320
NGROUP = NCHUNK // CPG
PER_TILE = NCHUNK * CHUNK
NNZ_PAD = PER_TILE * NS


def _mm_body(x_ref, w_ref, oa_ref, ob_ref):
    y = jnp.dot(x_ref[...], w_ref[...], preferred_element_type=jnp.float32)
    oa_ref[...] = y[:, :HALF]
    ob_ref[...] = y[:, HALF:]


def _matmul_halves(x0p, weight):
    rb = ROWS // 4
    return pl.pallas_call(
        _mm_body,
        grid=(4,),
        in_specs=[
            pl.BlockSpec((rb, D_IN), lambda i: (i, 0)),
            pl.BlockSpec((D_IN, D_OUT), lambda i: (0, 0)),
        ],
        out_specs=[
            pl.BlockSpec((rb, HALF), lambda i: (i, 0)),
            pl.BlockSpec((rb, HALF), lambda i: (i, 0)),
        ],
        out_shape=[
            jax.ShapeDtypeStruct((ROWS, HALF), jnp.float32),
            jax.ShapeDtypeStruct((ROWS, HALF), jnp.float32),
        ],
    )(x0p, weight)


def _hop(table, gidx, sidx, acc, bufs, gi_v, si_v, dsem, ssem, isem,
         zero_hbm):
    def dwait(sl):
        pltpu.make_async_copy(zero_hbm.at[pl.ds(0, CHUNK)], bufs.at[sl],
                              dsem.at[sl]).wait()

    def swait(sl):
        pltpu.make_async_copy(zero_hbm.at[pl.ds(0, CHUNK)], bufs.at[sl],
                              ssem.at[sl]).wait()

    def iwait():
        pltpu.make_async_copy(gidx.at[0], gi_v.at[0], isem).wait()

    pltpu.sync_copy(gidx.at[0], gi_v.at[0])
    pltpu.sync_copy(sidx.at[0], si_v.at[0])
    pltpu.async_copy(gidx.at[1], gi_v.at[1], isem)
    pltpu.async_copy(sidx.at[1], si_v.at[1], isem)
    for b in range(CPG):
        pltpu.async_copy(table.at[gi_v.at[0].at[b]], bufs.at[b], dsem.at[b])

    iwait()
    iwait()
    for b in range(CPG):
        dwait(b)
        pltpu.async_copy(bufs.at[b], acc.at[si_v.at[0].at[b]], ssem.at[b],
                         add=True)
    for b in range(CPG):
        pltpu.async_copy(table.at[gi_v.at[1].at[b]], bufs.at[CPG + b],
                         dsem.at[CPG + b])
    pltpu.async_copy(gidx.at[2], gi_v.at[2], isem)
    pltpu.async_copy(sidx.at[2], si_v.at[2], isem)

    def group(g, carry):
        p = g % 2
        q = (g + 1) % 2
        u = g % 3
        v = (g + 1) % 3
        w = (g + 2) % 3
        iwait()
        iwait()
        for b in range(CPG):
            sl = CPG * p + b
            dwait(sl)
            pltpu.async_copy(bufs.at[sl], acc.at[si_v.at[u].at[b]],
                             ssem.at[sl], add=True)
        for b in range(CPG):
            sl = CPG * q + b
            swait(sl)
            pltpu.async_copy(table.at[gi_v.at[v].at[b]], bufs.at[sl],
                             dsem.at[sl])
        gn = jnp.minimum(g + 2, NGROUP - 1)
        pltpu.async_copy(gidx.at[gn], gi_v.at[w], isem)
        pltpu.async_copy(sidx.at[gn], si_v.at[w], isem)
        return carry

    lax.fori_loop(1, NGROUP, group, 0)

    pl_ = (NGROUP - 1) % 2
    for b in range(CPG):
        swait(CPG * pl_ + b)
    for b in range(CPG):
        dwait(CPG * (1 - pl_) + b)
    iwait()
    iwait()


def _sc_body(xwa, xwb, node_hbm, edge_hbm, zero_hbm, out_a, out_b,
             gi_v, si_v, bufs, xwsp, acc_m, dsem, ssem, isem):
    c = lax.axis_index("c")
    s = lax.axis_index("s")
    r0 = s * ROWS_PER_TILE
    rows = pl.ds(r0, ROWS_PER_TILE)
    node_s = node_hbm.at[s]
    edge_s = edge_hbm.at[s]

    @pl.when(c == 0)
    def _():
        pltpu.sync_copy(xwa.at[rows], xwsp.at[rows])

    @pl.when(c == 1)
    def _():
        pltpu.sync_copy(xwb.at[rows], xwsp.at[rows])

    pltpu.sync_copy(zero_hbm.at[rows], acc_m.at[rows])
    plsc.subcore_barrier()

    _hop(xwsp, node_s, edge_s, acc_m, bufs, gi_v, si_v, dsem, ssem, isem,
         zero_hbm)
    plsc.subcore_barrier()

    pltpu.sync_copy(zero_hbm.at[rows], xwsp.at[rows])
    plsc.subcore_barrier()

    _hop(acc_m, edge_s, node_s, xwsp, bufs, gi_v, si_v, dsem, ssem, isem,
         zero_hbm)
    plsc.subcore_barrier()

    @pl.when(c == 0)
    def _():
        pltpu.sync_copy(xwsp.at[rows], out_a.at[rows])

    @pl.when(c == 1)
    def _():
        pltpu.sync_copy(xwsp.at[rows], out_b.at[rows])


_sc_call = pl.kernel(
    _sc_body,
    out_type=[
        jax.ShapeDtypeStruct((ROWS, HALF), jnp.float32),
        jax.ShapeDtypeStruct((ROWS, HALF), jnp.float32),
    ],
    mesh=plsc.VectorSubcoreMesh(core_axis_name="c", subcore_axis_name="s"),
    scratch_types=[
        pltpu.VMEM((3, CPG, CHUNK), jnp.int32),
        pltpu.VMEM((3, CPG, CHUNK), jnp.int32),
        pltpu.VMEM((NSLOT, CHUNK, HALF), jnp.float32),
        pltpu.VMEM_SHARED((ROWS, HALF), jnp.float32),
        pltpu.VMEM_SHARED((ROWS, HALF), jnp.float32),
        pltpu.SemaphoreType.DMA((NSLOT,)),
        pltpu.SemaphoreType.DMA((NSLOT,)),
        pltpu.SemaphoreType.DMA,
    ],
    compiler_params=pltpu.CompilerParams(use_tc_tiling_on_sc=False),
)


@jax.jit
def kernel(x_0, node_idx, edge_idx, weight):
    x0p = jnp.zeros((ROWS, D_IN), jnp.float32).at[:N_NODES].set(x_0)
    pad = jnp.full((NNZ_PAD - NNZ,), DUMMY, jnp.int32)
    node3 = jnp.concatenate([node_idx, pad]).reshape(NS, NGROUP, CPG, CHUNK)
    edge3 = jnp.concatenate([edge_idx, pad]).reshape(NS, NGROUP, CPG, CHUNK)
    zeros = jnp.zeros((ROWS, HALF), jnp.float32)
    xwa, xwb = _matmul_halves(x0p, weight)
    out_a, out_b = _sc_call(xwa, xwb, node3, edge3, zeros)
    return jnp.concatenate([out_a[:N_NODES], out_b[:N_NODES]], axis=1)


# speedup vs baseline: 1.8109x; 1.0063x over previous
import functools

import jax
import jax.numpy as jnp
from jax import lax
from jax.experimental import pallas as pl
from jax.experimental.pallas import tpu as pltpu
from jax.experimental.pallas import tpu_sc as plsc

N_NODES = 10000
N_EDGES = 10000
NNZ = 320000
D_IN = 128
D_OUT = 128
HALF = 64

NS = 16
ROWS = 10112
DUMMY = 10016
ROWS_PER_TILE = ROWS // NS
CHUNK = 128
CPG = 2
NSLOT = 2 * CPG
NCHUNK = 160
NGROUP = NCHUNK // CPG
PER_TILE = NCHUNK * CHUNK
NNZ_PAD = PER_TILE * NS


def _mm_body(x_ref, w_ref, oa_ref, ob_ref):
    y = jnp.dot(x_ref[...], w_ref[...], preferred_element_type=jnp.float32)
    oa_ref[...] = y[:, :HALF]
    ob_ref[...] = y[:, HALF:]


def _matmul_halves(x0p, weight):
    rb = ROWS // 4
    return pl.pallas_call(
        _mm_body,
        grid=(4,),
        in_specs=[
            pl.BlockSpec((rb, D_IN), lambda i: (i, 0)),
            pl.BlockSpec((D_IN, D_OUT), lambda i: (0, 0)),
        ],
        out_specs=[
            pl.BlockSpec((rb, HALF), lambda i: (i, 0)),
            pl.BlockSpec((rb, HALF), lambda i: (i, 0)),
        ],
        out_shape=[
            jax.ShapeDtypeStruct((ROWS, HALF), jnp.float32),
            jax.ShapeDtypeStruct((ROWS, HALF), jnp.float32),
        ],
    )(x0p, weight)


def _hop(table, gidx, sidx, acc, bufs, gi_v, si_v, dsem, ssem, isem,
         zero_hbm):
    def dwait(sl):
        pltpu.make_async_copy(zero_hbm.at[pl.ds(0, CHUNK)], bufs.at[sl],
                              dsem.at[sl]).wait()

    def swait(sl):
        pltpu.make_async_copy(zero_hbm.at[pl.ds(0, CHUNK)], bufs.at[sl],
                              ssem.at[sl]).wait()

    def iwait():
        pltpu.make_async_copy(gidx.at[0], gi_v.at[0], isem).wait()

    pltpu.sync_copy(gidx.at[0], gi_v.at[0])
    pltpu.sync_copy(sidx.at[0], si_v.at[0])
    pltpu.async_copy(gidx.at[1], gi_v.at[1], isem)
    pltpu.async_copy(sidx.at[1], si_v.at[1], isem)
    for b in range(CPG):
        pltpu.async_copy(table.at[gi_v.at[0].at[b]], bufs.at[b], dsem.at[b])

    iwait()
    iwait()
    for b in range(CPG):
        dwait(b)
        pltpu.async_copy(bufs.at[b], acc.at[si_v.at[0].at[b]], ssem.at[b],
                         add=True)
    for b in range(CPG):
        pltpu.async_copy(table.at[gi_v.at[1].at[b]], bufs.at[CPG + b],
                         dsem.at[CPG + b])
    pltpu.async_copy(gidx.at[2], gi_v.at[2], isem)
    pltpu.async_copy(sidx.at[2], si_v.at[2], isem)

    def group(g, carry):
        p = g % 2
        q = (g + 1) % 2
        u = g % 3
        v = (g + 1) % 3
        w = (g + 2) % 3
        iwait()
        iwait()
        for b in range(CPG):
            sl = CPG * p + b
            dwait(sl)
            pltpu.async_copy(bufs.at[sl], acc.at[si_v.at[u].at[b]],
                             ssem.at[sl], add=True)
        for b in range(CPG):
            sl = CPG * q + b
            swait(sl)
            pltpu.async_copy(table.at[gi_v.at[v].at[b]], bufs.at[sl],
                             dsem.at[sl])
        gn = jnp.minimum(g + 2, NGROUP - 1)
        pltpu.async_copy(gidx.at[gn], gi_v.at[w], isem)
        pltpu.async_copy(sidx.at[gn], si_v.at[w], isem)
        return carry

    lax.fori_loop(1, NGROUP, group, 0)

    pl_ = (NGROUP - 1) % 2
    for b in range(CPG):
        swait(CPG * pl_ + b)
    for b in range(CPG):
        dwait(CPG * (1 - pl_) + b)
    iwait()
    iwait()


def _sc_body(xwa, xwb, node_hbm, edge_hbm, zero_hbm, out_a, out_b,
             gi_v, si_v, bufs, xwsp, acc_m, dsem, ssem, isem):
    c = lax.axis_index("c")
    s = lax.axis_index("s")
    r0 = s * ROWS_PER_TILE
    rows = pl.ds(r0, ROWS_PER_TILE)
    node_s = node_hbm.at[s]
    edge_s = edge_hbm.at[s]

    @pl.when(c == 0)
    def _():
        pltpu.sync_copy(xwa.at[rows], xwsp.at[rows])

    @pl.when(c == 1)
    def _():
        pltpu.sync_copy(xwb.at[rows], xwsp.at[rows])

    pltpu.sync_copy(zero_hbm.at[rows], acc_m.at[rows])
    plsc.subcore_barrier()

    _hop(xwsp, node_s, edge_s, acc_m, bufs, gi_v, si_v, dsem, ssem, isem,
         zero_hbm)
    plsc.subcore_barrier()

    pltpu.sync_copy(zero_hbm.at[rows], xwsp.at[rows])
    plsc.subcore_barrier()

    _hop(acc_m, edge_s, node_s, xwsp, bufs, gi_v, si_v, dsem, ssem, isem,
         zero_hbm)
    plsc.subcore_barrier()

    @pl.when(c == 0)
    def _():
        pltpu.sync_copy(xwsp.at[rows], out_a.at[rows])

    @pl.when(c == 1)
    def _():
        pltpu.sync_copy(xwsp.at[rows], out_b.at[rows])


_sc_call = pl.kernel(
    _sc_body,
    out_type=[
        jax.ShapeDtypeStruct((ROWS, HALF), jnp.float32),
        jax.ShapeDtypeStruct((ROWS, HALF), jnp.float32),
    ],
    mesh=plsc.VectorSubcoreMesh(core_axis_name="c", subcore_axis_name="s"),
    scratch_types=[
        pltpu.VMEM((3, CPG, CHUNK), jnp.int32),
        pltpu.VMEM((3, CPG, CHUNK), jnp.int32),
        pltpu.VMEM((NSLOT, CHUNK, HALF), jnp.float32),
        pltpu.VMEM_SHARED((ROWS, HALF), jnp.float32),
        pltpu.VMEM_SHARED((ROWS, HALF), jnp.float32),
        pltpu.SemaphoreType.DMA((NSLOT,)),
        pltpu.SemaphoreType.DMA((NSLOT,)),
        pltpu.SemaphoreType.DMA,
    ],
    compiler_params=pltpu.CompilerParams(use_tc_tiling_on_sc=False),
)


@jax.jit
def kernel(x_0, node_idx, edge_idx, weight):
    x0p = jnp.zeros((ROWS, D_IN), jnp.float32).at[:N_NODES].set(x_0)
    pad = jnp.full((NNZ_PAD - NNZ,), DUMMY, jnp.int32)
    node3 = jnp.concatenate([node_idx, pad]).reshape(NS, NGROUP, CPG, CHUNK)
    edge3 = jnp.concatenate([edge_idx, pad]).reshape(NS, NGROUP, CPG, CHUNK)
    zeros = jnp.zeros((ROWS, HALF), jnp.float32)
    xwa, xwb = _matmul_halves(x0p, weight)
    out_a, out_b = _sc_call(xwa, xwb, node3, edge3, zeros)
    return jnp.concatenate([out_a[:N_NODES], out_b[:N_NODES]], axis=1)


# speedup vs baseline: 1.9050x; 1.0519x over previous
import functools

import jax
import jax.numpy as jnp
from jax import lax
from jax.experimental import pallas as pl
from jax.experimental.pallas import tpu as pltpu
from jax.experimental.pallas import tpu_sc as plsc

N_NODES = 10000
N_EDGES = 10000
NNZ = 320000
D_IN = 128
D_OUT = 128
HALF = 64

NS = 16
ROWS = 10112
DUMMY = 10016
ROWS_PER_TILE = ROWS // NS
CHUNK = 128
CPG = 2
NSLOT = 2 * CPG
NCHUNK = 160
NGROUP = NCHUNK // CPG
PER_TILE = NCHUNK * CHUNK
NNZ_PAD = PER_TILE * NS


def _mm_body(x_ref, w_ref, oa_ref, ob_ref):
    y = jnp.dot(x_ref[...], w_ref[...], preferred_element_type=jnp.float32)
    oa_ref[...] = y[:, :HALF]
    ob_ref[...] = y[:, HALF:]


def _matmul_halves(x_0, weight):
    rb = ROWS // 4
    return pl.pallas_call(
        _mm_body,
        grid=(4,),
        in_specs=[
            pl.BlockSpec((rb, D_IN), lambda i: (i, 0)),
            pl.BlockSpec((D_IN, D_OUT), lambda i: (0, 0)),
        ],
        out_specs=[
            pl.BlockSpec((rb, HALF), lambda i: (i, 0)),
            pl.BlockSpec((rb, HALF), lambda i: (i, 0)),
        ],
        out_shape=[
            jax.ShapeDtypeStruct((ROWS, HALF), jnp.float32),
            jax.ShapeDtypeStruct((ROWS, HALF), jnp.float32),
        ],
    )(x_0, weight)


def _hop(table, gidx, sidx, acc, bufs, gi_v, si_v, dsem, ssem, isem,
         zero_hbm):
    def dwait(sl):
        pltpu.make_async_copy(zero_hbm.at[pl.ds(0, CHUNK)], bufs.at[sl],
                              dsem.at[sl]).wait()

    def swait(sl):
        pltpu.make_async_copy(zero_hbm.at[pl.ds(0, CHUNK)], bufs.at[sl],
                              ssem.at[sl]).wait()

    def iwait():
        pltpu.make_async_copy(gidx.at[0], gi_v.at[0], isem).wait()

    pltpu.sync_copy(gidx.at[0], gi_v.at[0])
    pltpu.sync_copy(sidx.at[0], si_v.at[0])
    pltpu.async_copy(gidx.at[1], gi_v.at[1], isem)
    pltpu.async_copy(sidx.at[1], si_v.at[1], isem)
    for b in range(CPG):
        pltpu.async_copy(table.at[gi_v.at[0].at[b]], bufs.at[b], dsem.at[b])

    iwait()
    iwait()
    for b in range(CPG):
        dwait(b)
        pltpu.async_copy(bufs.at[b], acc.at[si_v.at[0].at[b]], ssem.at[b],
                         add=True)
    for b in range(CPG):
        pltpu.async_copy(table.at[gi_v.at[1].at[b]], bufs.at[CPG + b],
                         dsem.at[CPG + b])
    pltpu.async_copy(gidx.at[2], gi_v.at[2], isem)
    pltpu.async_copy(sidx.at[2], si_v.at[2], isem)

    def group(g, carry):
        p = g % 2
        q = (g + 1) % 2
        u = g % 3
        v = (g + 1) % 3
        w = (g + 2) % 3
        iwait()
        iwait()
        for b in range(CPG):
            sl = CPG * p + b
            dwait(sl)
            pltpu.async_copy(bufs.at[sl], acc.at[si_v.at[u].at[b]],
                             ssem.at[sl], add=True)
        for b in range(CPG):
            sl = CPG * q + b
            swait(sl)
            pltpu.async_copy(table.at[gi_v.at[v].at[b]], bufs.at[sl],
                             dsem.at[sl])
        gn = jnp.minimum(g + 2, NGROUP - 1)
        pltpu.async_copy(gidx.at[gn], gi_v.at[w], isem)
        pltpu.async_copy(sidx.at[gn], si_v.at[w], isem)
        return carry

    lax.fori_loop(1, NGROUP, group, 0)

    pl_ = (NGROUP - 1) % 2
    for b in range(CPG):
        swait(CPG * pl_ + b)
    for b in range(CPG):
        dwait(CPG * (1 - pl_) + b)
    iwait()
    iwait()


def _sc_body(xwa, xwb, node_hbm, edge_hbm, zero_hbm, out,
             gi_v, si_v, bufs, xwsp, acc_m, dsem, ssem, isem):
    c = lax.axis_index("c")
    s = lax.axis_index("s")
    r0 = s * ROWS_PER_TILE
    rows = pl.ds(r0, ROWS_PER_TILE)
    node_s = node_hbm.at[s]
    edge_s = edge_hbm.at[s]

    @pl.when(c == 0)
    def _():
        pltpu.sync_copy(xwa.at[rows], xwsp.at[rows])

    @pl.when(c == 1)
    def _():
        pltpu.sync_copy(xwb.at[rows], xwsp.at[rows])

    pltpu.sync_copy(zero_hbm.at[rows], acc_m.at[rows])
    plsc.subcore_barrier()

    _hop(xwsp, node_s, edge_s, acc_m, bufs, gi_v, si_v, dsem, ssem, isem,
         zero_hbm)
    plsc.subcore_barrier()

    pltpu.sync_copy(zero_hbm.at[rows], xwsp.at[rows])
    plsc.subcore_barrier()

    _hop(acc_m, edge_s, node_s, xwsp, bufs, gi_v, si_v, dsem, ssem, isem,
         zero_hbm)
    plsc.subcore_barrier()

    wrows = pl.ds(s * (N_NODES // NS), N_NODES // NS)

    @pl.when(c == 0)
    def _():
        pltpu.sync_copy(xwsp.at[wrows], out.at[wrows, pl.ds(0, HALF)])

    @pl.when(c == 1)
    def _():
        pltpu.sync_copy(xwsp.at[wrows], out.at[wrows, pl.ds(HALF, HALF)])


_sc_call = pl.kernel(
    _sc_body,
    out_type=jax.ShapeDtypeStruct((N_NODES, D_OUT), jnp.float32),
    mesh=plsc.VectorSubcoreMesh(core_axis_name="c", subcore_axis_name="s"),
    scratch_types=[
        pltpu.VMEM((3, CPG, CHUNK), jnp.int32),
        pltpu.VMEM((3, CPG, CHUNK), jnp.int32),
        pltpu.VMEM((NSLOT, CHUNK, HALF), jnp.float32),
        pltpu.VMEM_SHARED((ROWS, HALF), jnp.float32),
        pltpu.VMEM_SHARED((ROWS, HALF), jnp.float32),
        pltpu.SemaphoreType.DMA((NSLOT,)),
        pltpu.SemaphoreType.DMA((NSLOT,)),
        pltpu.SemaphoreType.DMA,
    ],
    compiler_params=pltpu.CompilerParams(use_tc_tiling_on_sc=False),
)


@jax.jit
def kernel(x_0, node_idx, edge_idx, weight):
    pad = jnp.full((NNZ_PAD - NNZ,), DUMMY, jnp.int32)
    node3 = jnp.concatenate([node_idx, pad]).reshape(NS, NGROUP, CPG, CHUNK)
    edge3 = jnp.concatenate([edge_idx, pad]).reshape(NS, NGROUP, CPG, CHUNK)
    zeros = jnp.zeros((ROWS, HALF), jnp.float32)
    xwa, xwb = _matmul_halves(x_0, weight)
    return _sc_call(xwa, xwb, node3, edge3, zeros)


# speedup vs baseline: 2.4719x; 1.2976x over previous
import functools

import jax
import jax.numpy as jnp
from jax import lax
from jax.experimental import pallas as pl
from jax.experimental.pallas import tpu as pltpu
from jax.experimental.pallas import tpu_sc as plsc

N_NODES = 10000
N_EDGES = 10000
NNZ = 320000
D_IN = 128
D_OUT = 128
HALF = 64

NS = 16
ROWS = 10112
DUMMY = 10016
ROWS_PER_TILE = ROWS // NS
CHUNK = 128
CPG = 2
NSLOT = 2 * CPG
NCHUNK = 160
NGROUP = NCHUNK // CPG
PER_TILE = NCHUNK * CHUNK
NNZ_PAD = PER_TILE * NS


def _mm_body(x_ref, w_ref, oa_ref, ob_ref):
    y = jnp.dot(x_ref[...], w_ref[...], preferred_element_type=jnp.float32)
    oa_ref[...] = y[:, :HALF]
    ob_ref[...] = y[:, HALF:]


def _matmul_halves(x_0, weight):
    rb = ROWS // 4
    return pl.pallas_call(
        _mm_body,
        grid=(4,),
        in_specs=[
            pl.BlockSpec((rb, D_IN), lambda i: (i, 0)),
            pl.BlockSpec((D_IN, D_OUT), lambda i: (0, 0)),
        ],
        out_specs=[
            pl.BlockSpec((rb, HALF), lambda i: (i, 0)),
            pl.BlockSpec((rb, HALF), lambda i: (i, 0)),
        ],
        out_shape=[
            jax.ShapeDtypeStruct((ROWS, HALF), jnp.float32),
            jax.ShapeDtypeStruct((ROWS, HALF), jnp.float32),
        ],
    )(x_0, weight)


def _hop(table, gidx, sidx, acc, bufs, gi_v, si_v, dsem, ssem, isem,
         zero_hbm):
    def dwait(sl):
        pltpu.make_async_copy(zero_hbm.at[pl.ds(0, CHUNK)], bufs.at[sl],
                              dsem.at[sl]).wait()

    def swait(sl):
        pltpu.make_async_copy(zero_hbm.at[pl.ds(0, CHUNK)], bufs.at[sl],
                              ssem.at[sl]).wait()

    def iwait():
        pltpu.make_async_copy(gidx.at[0], gi_v.at[0], isem).wait()

    pltpu.sync_copy(gidx.at[0], gi_v.at[0])
    pltpu.sync_copy(sidx.at[0], si_v.at[0])
    pltpu.async_copy(gidx.at[1], gi_v.at[1], isem)
    pltpu.async_copy(sidx.at[1], si_v.at[1], isem)
    for b in range(CPG):
        pltpu.async_copy(table.at[gi_v.at[0].at[b]], bufs.at[b], dsem.at[b])

    iwait()
    iwait()
    for b in range(CPG):
        dwait(b)
        pltpu.async_copy(bufs.at[b], acc.at[si_v.at[0].at[b]], ssem.at[b],
                         add=True)
    for b in range(CPG):
        pltpu.async_copy(table.at[gi_v.at[1].at[b]], bufs.at[CPG + b],
                         dsem.at[CPG + b])
    pltpu.async_copy(gidx.at[2], gi_v.at[2], isem)
    pltpu.async_copy(sidx.at[2], si_v.at[2], isem)

    def group(g, carry):
        p = g % 2
        q = (g + 1) % 2
        u = g % 3
        v = (g + 1) % 3
        w = (g + 2) % 3
        iwait()
        iwait()
        for b in range(CPG):
            sl = CPG * p + b
            sl2 = CPG * q + b
            dwait(sl)
            pltpu.async_copy(bufs.at[sl], acc.at[si_v.at[u].at[b]],
                             ssem.at[sl], add=True)
            swait(sl2)
            pltpu.async_copy(table.at[gi_v.at[v].at[b]], bufs.at[sl2],
                             dsem.at[sl2])
        gn = jnp.minimum(g + 2, NGROUP - 1)
        pltpu.async_copy(gidx.at[gn], gi_v.at[w], isem)
        pltpu.async_copy(sidx.at[gn], si_v.at[w], isem)
        return carry

    lax.fori_loop(1, NGROUP, group, 0)

    pl_ = (NGROUP - 1) % 2
    for b in range(CPG):
        swait(CPG * pl_ + b)
    for b in range(CPG):
        dwait(CPG * (1 - pl_) + b)
    iwait()
    iwait()


def _sc_body(xwa, xwb, node_hbm, edge_hbm, zero_hbm, out,
             gi_v, si_v, bufs, xwsp, acc_m, dsem, ssem, isem):
    c = lax.axis_index("c")
    s = lax.axis_index("s")
    r0 = s * ROWS_PER_TILE
    rows = pl.ds(r0, ROWS_PER_TILE)
    node_s = node_hbm.at[s]
    edge_s = edge_hbm.at[s]

    @pl.when(c == 0)
    def _():
        pltpu.sync_copy(xwa.at[rows], xwsp.at[rows])

    @pl.when(c == 1)
    def _():
        pltpu.sync_copy(xwb.at[rows], xwsp.at[rows])

    pltpu.sync_copy(zero_hbm.at[rows], acc_m.at[rows])
    plsc.subcore_barrier()

    _hop(xwsp, node_s, edge_s, acc_m, bufs, gi_v, si_v, dsem, ssem, isem,
         zero_hbm)
    plsc.subcore_barrier()

    pltpu.sync_copy(zero_hbm.at[rows], xwsp.at[rows])
    plsc.subcore_barrier()

    _hop(acc_m, edge_s, node_s, xwsp, bufs, gi_v, si_v, dsem, ssem, isem,
         zero_hbm)
    plsc.subcore_barrier()

    wrows = pl.ds(s * (N_NODES // NS), N_NODES // NS)

    @pl.when(c == 0)
    def _():
        pltpu.sync_copy(xwsp.at[wrows], out.at[wrows, pl.ds(0, HALF)])

    @pl.when(c == 1)
    def _():
        pltpu.sync_copy(xwsp.at[wrows], out.at[wrows, pl.ds(HALF, HALF)])


_sc_call = pl.kernel(
    _sc_body,
    out_type=jax.ShapeDtypeStruct((N_NODES, D_OUT), jnp.float32),
    mesh=plsc.VectorSubcoreMesh(core_axis_name="c", subcore_axis_name="s"),
    scratch_types=[
        pltpu.VMEM((3, CPG, CHUNK), jnp.int32),
        pltpu.VMEM((3, CPG, CHUNK), jnp.int32),
        pltpu.VMEM((NSLOT, CHUNK, HALF), jnp.float32),
        pltpu.VMEM_SHARED((ROWS, HALF), jnp.float32),
        pltpu.VMEM_SHARED((ROWS, HALF), jnp.float32),
        pltpu.SemaphoreType.DMA((NSLOT,)),
        pltpu.SemaphoreType.DMA((NSLOT,)),
        pltpu.SemaphoreType.DMA,
    ],
    compiler_params=pltpu.CompilerParams(use_tc_tiling_on_sc=False),
)


@jax.jit
def kernel(x_0, node_idx, edge_idx, weight):
    pad = jnp.full((NNZ_PAD - NNZ,), DUMMY, jnp.int32)
    node3 = jnp.concatenate([node_idx, pad]).reshape(NS, NGROUP, CPG, CHUNK)
    edge3 = jnp.concatenate([edge_idx, pad]).reshape(NS, NGROUP, CPG, CHUNK)
    zeros = jnp.zeros((ROWS, HALF), jnp.float32)
    xwa, xwb = _matmul_halves(x_0, weight)
    return _sc_call(xwa, xwb, node3, edge3, zeros)


# speedup vs baseline: 2.4766x; 1.0019x over previous
import functools

import jax
import jax.numpy as jnp
from jax import lax
from jax.experimental import pallas as pl
from jax.experimental.pallas import tpu as pltpu
from jax.experimental.pallas import tpu_sc as plsc

N_NODES = 10000
N_EDGES = 10000
NNZ = 320000
D_IN = 128
D_OUT = 128
HALF = 64

NS = 16
ROWS = 10112
DUMMY = 10016
ROWS_PER_TILE = ROWS // NS
CHUNK = 128
CPG = 2
NSLOT = 2 * CPG
NCHUNK = 160
NGROUP = NCHUNK // CPG
PER_TILE = NCHUNK * CHUNK
NNZ_PAD = PER_TILE * NS


def _mm_body(x_ref, w_ref, oa_ref, ob_ref):
    y = jnp.dot(x_ref[...], w_ref[...], preferred_element_type=jnp.float32)
    oa_ref[...] = y[:, :HALF]
    ob_ref[...] = y[:, HALF:]


def _matmul_halves(x_0, weight):
    rb = ROWS // 4
    return pl.pallas_call(
        _mm_body,
        grid=(4,),
        in_specs=[
            pl.BlockSpec((rb, D_IN), lambda i: (i, 0)),
            pl.BlockSpec((D_IN, D_OUT), lambda i: (0, 0)),
        ],
        out_specs=[
            pl.BlockSpec((rb, HALF), lambda i: (i, 0)),
            pl.BlockSpec((rb, HALF), lambda i: (i, 0)),
        ],
        out_shape=[
            jax.ShapeDtypeStruct((ROWS, HALF), jnp.float32),
            jax.ShapeDtypeStruct((ROWS, HALF), jnp.float32),
        ],
    )(x_0, weight)


def _hop(table, gidx, sidx, acc, bufs, gi_v, si_v, dsem, ssem, isem,
         zero_hbm):
    def dwait(sl):
        pltpu.make_async_copy(table.at[gi_v.at[0].at[0]], bufs.at[sl],
                              dsem.at[sl]).wait()

    def swait(sl):
        pltpu.make_async_copy(bufs.at[sl], acc.at[si_v.at[0].at[0]],
                              ssem.at[sl]).wait()

    def iwait():
        pltpu.make_async_copy(gidx.at[0], gi_v.at[0], isem).wait()

    pltpu.sync_copy(gidx.at[0], gi_v.at[0])
    pltpu.sync_copy(sidx.at[0], si_v.at[0])
    pltpu.async_copy(gidx.at[1], gi_v.at[1], isem)
    pltpu.async_copy(sidx.at[1], si_v.at[1], isem)
    for b in range(CPG):
        pltpu.async_copy(table.at[gi_v.at[0].at[b]], bufs.at[b], dsem.at[b])

    iwait()
    iwait()
    for b in range(CPG):
        dwait(b)
        pltpu.async_copy(bufs.at[b], acc.at[si_v.at[0].at[b]], ssem.at[b],
                         add=True)
    for b in range(CPG):
        pltpu.async_copy(table.at[gi_v.at[1].at[b]], bufs.at[CPG + b],
                         dsem.at[CPG + b])
    pltpu.async_copy(gidx.at[2], gi_v.at[2], isem)
    pltpu.async_copy(sidx.at[2], si_v.at[2], isem)

    def group(g, carry):
        p = g % 2
        q = (g + 1) % 2
        u = g % 3
        v = (g + 1) % 3
        w = (g + 2) % 3
        iwait()
        iwait()
        for b in range(CPG):
            sl = CPG * p + b
            sl2 = CPG * q + b
            dwait(sl)
            pltpu.async_copy(bufs.at[sl], acc.at[si_v.at[u].at[b]],
                             ssem.at[sl], add=True)
            swait(sl2)
            pltpu.async_copy(table.at[gi_v.at[v].at[b]], bufs.at[sl2],
                             dsem.at[sl2])
        gn = jnp.minimum(g + 2, NGROUP - 1)
        pltpu.async_copy(gidx.at[gn], gi_v.at[w], isem)
        pltpu.async_copy(sidx.at[gn], si_v.at[w], isem)
        return carry

    lax.fori_loop(1, NGROUP, group, 0)

    pl_ = (NGROUP - 1) % 2
    for b in range(CPG):
        swait(CPG * pl_ + b)
    for b in range(CPG):
        dwait(CPG * (1 - pl_) + b)
    iwait()
    iwait()


def _sc_body(xwa, xwb, node_hbm, edge_hbm, zero_hbm, out,
             gi_v, si_v, bufs, xwsp, acc_m, dsem, ssem, isem):
    c = lax.axis_index("c")
    s = lax.axis_index("s")
    r0 = s * ROWS_PER_TILE
    rows = pl.ds(r0, ROWS_PER_TILE)
    node_s = node_hbm.at[s]
    edge_s = edge_hbm.at[s]

    @pl.when(c == 0)
    def _():
        pltpu.sync_copy(xwa.at[rows], xwsp.at[rows])

    @pl.when(c == 1)
    def _():
        pltpu.sync_copy(xwb.at[rows], xwsp.at[rows])

    pltpu.sync_copy(zero_hbm.at[rows], acc_m.at[rows])
    plsc.subcore_barrier()

    _hop(xwsp, node_s, edge_s, acc_m, bufs, gi_v, si_v, dsem, ssem, isem,
         zero_hbm)
    plsc.subcore_barrier()

    pltpu.sync_copy(zero_hbm.at[rows], xwsp.at[rows])
    plsc.subcore_barrier()

    _hop(acc_m, edge_s, node_s, xwsp, bufs, gi_v, si_v, dsem, ssem, isem,
         zero_hbm)
    plsc.subcore_barrier()

    wrows = pl.ds(s * (N_NODES // NS), N_NODES // NS)

    @pl.when(c == 0)
    def _():
        pltpu.sync_copy(xwsp.at[wrows], out.at[wrows, pl.ds(0, HALF)])

    @pl.when(c == 1)
    def _():
        pltpu.sync_copy(xwsp.at[wrows], out.at[wrows, pl.ds(HALF, HALF)])


_sc_call = pl.kernel(
    _sc_body,
    out_type=jax.ShapeDtypeStruct((N_NODES, D_OUT), jnp.float32),
    mesh=plsc.VectorSubcoreMesh(core_axis_name="c", subcore_axis_name="s"),
    scratch_types=[
        pltpu.VMEM((3, CPG, CHUNK), jnp.int32),
        pltpu.VMEM((3, CPG, CHUNK), jnp.int32),
        pltpu.VMEM((NSLOT, CHUNK, HALF), jnp.float32),
        pltpu.VMEM_SHARED((ROWS, HALF), jnp.float32),
        pltpu.VMEM_SHARED((ROWS, HALF), jnp.float32),
        pltpu.SemaphoreType.DMA((NSLOT,)),
        pltpu.SemaphoreType.DMA((NSLOT,)),
        pltpu.SemaphoreType.DMA,
    ],
    compiler_params=pltpu.CompilerParams(use_tc_tiling_on_sc=False),
)


@jax.jit
def kernel(x_0, node_idx, edge_idx, weight):
    pad = jnp.full((NNZ_PAD - NNZ,), DUMMY, jnp.int32)
    node3 = jnp.concatenate([node_idx, pad]).reshape(NS, NGROUP, CPG, CHUNK)
    edge3 = jnp.concatenate([edge_idx, pad]).reshape(NS, NGROUP, CPG, CHUNK)
    zeros = jnp.zeros((ROWS, HALF), jnp.float32)
    xwa, xwb = _matmul_halves(x_0, weight)
    return _sc_call(xwa, xwb, node3, edge3, zeros)


# speedup vs baseline: 2.4819x; 1.0021x over previous
import functools

import jax
import jax.numpy as jnp
from jax import lax
from jax.experimental import pallas as pl
from jax.experimental.pallas import tpu as pltpu
from jax.experimental.pallas import tpu_sc as plsc

N_NODES = 10000
N_EDGES = 10000
NNZ = 320000
D_IN = 128
D_OUT = 128
HALF = 64

NS = 16
ROWS = 10112
DUMMY = 10016
ROWS_PER_TILE = ROWS // NS
CHUNK = 64
CPG = 4
NSLOT = 2 * CPG
NCHUNK = 320
NGROUP = NCHUNK // CPG
PER_TILE = NCHUNK * CHUNK
NNZ_PAD = PER_TILE * NS


def _mm_body(x_ref, w_ref, oa_ref, ob_ref):
    y = jnp.dot(x_ref[...], w_ref[...], preferred_element_type=jnp.float32)
    oa_ref[...] = y[:, :HALF]
    ob_ref[...] = y[:, HALF:]


def _matmul_halves(x_0, weight):
    rb = ROWS // 4
    return pl.pallas_call(
        _mm_body,
        grid=(4,),
        in_specs=[
            pl.BlockSpec((rb, D_IN), lambda i: (i, 0)),
            pl.BlockSpec((D_IN, D_OUT), lambda i: (0, 0)),
        ],
        out_specs=[
            pl.BlockSpec((rb, HALF), lambda i: (i, 0)),
            pl.BlockSpec((rb, HALF), lambda i: (i, 0)),
        ],
        out_shape=[
            jax.ShapeDtypeStruct((ROWS, HALF), jnp.float32),
            jax.ShapeDtypeStruct((ROWS, HALF), jnp.float32),
        ],
    )(x_0, weight)


def _hop(table, gidx, sidx, acc, bufs, gi_v, si_v, dsem, ssem, isem,
         zero_hbm):
    def dwait(sl):
        pltpu.make_async_copy(table.at[gi_v.at[0].at[0]], bufs.at[sl],
                              dsem.at[sl]).wait()

    def swait(sl):
        pltpu.make_async_copy(bufs.at[sl], acc.at[si_v.at[0].at[0]],
                              ssem.at[sl]).wait()

    def iwait():
        pltpu.make_async_copy(gidx.at[0], gi_v.at[0], isem).wait()

    pltpu.sync_copy(gidx.at[0], gi_v.at[0])
    pltpu.sync_copy(sidx.at[0], si_v.at[0])
    pltpu.async_copy(gidx.at[1], gi_v.at[1], isem)
    pltpu.async_copy(sidx.at[1], si_v.at[1], isem)
    for b in range(CPG):
        pltpu.async_copy(table.at[gi_v.at[0].at[b]], bufs.at[b], dsem.at[b])

    iwait()
    iwait()
    for b in range(CPG):
        dwait(b)
        pltpu.async_copy(bufs.at[b], acc.at[si_v.at[0].at[b]], ssem.at[b],
                         add=True)
    for b in range(CPG):
        pltpu.async_copy(table.at[gi_v.at[1].at[b]], bufs.at[CPG + b],
                         dsem.at[CPG + b])
    pltpu.async_copy(gidx.at[2], gi_v.at[2], isem)
    pltpu.async_copy(sidx.at[2], si_v.at[2], isem)

    def group(g, carry):
        p = g % 2
        q = (g + 1) % 2
        u = g % 3
        v = (g + 1) % 3
        w = (g + 2) % 3
        iwait()
        iwait()
        for b in range(CPG):
            sl = CPG * p + b
            sl2 = CPG * q + b
            dwait(sl)
            pltpu.async_copy(bufs.at[sl], acc.at[si_v.at[u].at[b]],
                             ssem.at[sl], add=True)
            swait(sl2)
            pltpu.async_copy(table.at[gi_v.at[v].at[b]], bufs.at[sl2],
                             dsem.at[sl2])
        gn = jnp.minimum(g + 2, NGROUP - 1)
        pltpu.async_copy(gidx.at[gn], gi_v.at[w], isem)
        pltpu.async_copy(sidx.at[gn], si_v.at[w], isem)
        return carry

    lax.fori_loop(1, NGROUP, group, 0)

    pl_ = (NGROUP - 1) % 2
    for b in range(CPG):
        swait(CPG * pl_ + b)
    for b in range(CPG):
        dwait(CPG * (1 - pl_) + b)
    iwait()
    iwait()


def _sc_body(xwa, xwb, node_hbm, edge_hbm, zero_hbm, out,
             gi_v, si_v, bufs, xwsp, acc_m, dsem, ssem, isem):
    c = lax.axis_index("c")
    s = lax.axis_index("s")
    r0 = s * ROWS_PER_TILE
    rows = pl.ds(r0, ROWS_PER_TILE)
    node_s = node_hbm.at[s]
    edge_s = edge_hbm.at[s]

    @pl.when(c == 0)
    def _():
        pltpu.sync_copy(xwa.at[rows], xwsp.at[rows])

    @pl.when(c == 1)
    def _():
        pltpu.sync_copy(xwb.at[rows], xwsp.at[rows])

    pltpu.sync_copy(zero_hbm.at[rows], acc_m.at[rows])
    plsc.subcore_barrier()

    _hop(xwsp, node_s, edge_s, acc_m, bufs, gi_v, si_v, dsem, ssem, isem,
         zero_hbm)
    plsc.subcore_barrier()

    pltpu.sync_copy(zero_hbm.at[rows], xwsp.at[rows])
    plsc.subcore_barrier()

    _hop(acc_m, edge_s, node_s, xwsp, bufs, gi_v, si_v, dsem, ssem, isem,
         zero_hbm)
    plsc.subcore_barrier()

    wrows = pl.ds(s * (N_NODES // NS), N_NODES // NS)

    @pl.when(c == 0)
    def _():
        pltpu.sync_copy(xwsp.at[wrows], out.at[wrows, pl.ds(0, HALF)])

    @pl.when(c == 1)
    def _():
        pltpu.sync_copy(xwsp.at[wrows], out.at[wrows, pl.ds(HALF, HALF)])


_sc_call = pl.kernel(
    _sc_body,
    out_type=jax.ShapeDtypeStruct((N_NODES, D_OUT), jnp.float32),
    mesh=plsc.VectorSubcoreMesh(core_axis_name="c", subcore_axis_name="s"),
    scratch_types=[
        pltpu.VMEM((3, CPG, CHUNK), jnp.int32),
        pltpu.VMEM((3, CPG, CHUNK), jnp.int32),
        pltpu.VMEM((NSLOT, CHUNK, HALF), jnp.float32),
        pltpu.VMEM_SHARED((ROWS, HALF), jnp.float32),
        pltpu.VMEM_SHARED((ROWS, HALF), jnp.float32),
        pltpu.SemaphoreType.DMA((NSLOT,)),
        pltpu.SemaphoreType.DMA((NSLOT,)),
        pltpu.SemaphoreType.DMA,
    ],
    compiler_params=pltpu.CompilerParams(use_tc_tiling_on_sc=False),
)


@jax.jit
def kernel(x_0, node_idx, edge_idx, weight):
    pad = jnp.full((NNZ_PAD - NNZ,), DUMMY, jnp.int32)
    node3 = jnp.concatenate([node_idx, pad]).reshape(NS, NGROUP, CPG, CHUNK)
    edge3 = jnp.concatenate([edge_idx, pad]).reshape(NS, NGROUP, CPG, CHUNK)
    zeros = jnp.zeros((ROWS, HALF), jnp.float32)
    xwa, xwb = _matmul_halves(x_0, weight)
    return _sc_call(xwa, xwb, node3, edge3, zeros)
